# Initial kernel scaffold; baseline (speedup 1.0000x reference)
import jax
import jax.numpy as jnp
from jax.experimental import pallas as pl


def kernel(input, weights):
    raise NotImplementedError("write your pallas kernel here")


# speedup vs baseline: 2.1103x; 2.1103x over previous
import functools

import jax
import jax.numpy as jnp
from jax import lax
from jax.experimental import pallas as pl
from jax.experimental.pallas import tpu as pltpu
from jax.experimental.pallas import tpu_sc as plsc

L = 16


def _take(v, idx):
    return v.at[idx].get(mode="promise_in_bounds")


def _cumsum16(m):
    lanes = lax.iota(jnp.int32, L)
    v = m
    for k in (1, 2, 4, 8):
        shifted = _take(v, jnp.maximum(lanes - k, 0))
        ind = jnp.minimum(jnp.maximum(lanes - (k - 1), 0), 1)
        v = v + shifted * ind
    return v


def _bcast_last(v):
    return _take(v, jnp.full((L,), L - 1, jnp.int32))


def _emb_lookup(inp_flat, weights, *, rows_per_worker, seq_len, chunk):
    n_workers = inp_flat.shape[0] // rows_per_worker
    segs_per_row = seq_len // rows_per_worker
    n_chunks = rows_per_worker // chunk
    emb_dim = weights.shape[1]
    mesh = plsc.VectorSubcoreMesh(core_axis_name="c", subcore_axis_name="s")

    @functools.partial(
        pl.kernel,
        out_type=jax.ShapeDtypeStruct((inp_flat.shape[0], emb_dim), jnp.float32),
        mesh=mesh,
        scratch_types=[
            pltpu.VMEM((seq_len,), jnp.int32),
            pltpu.VMEM((rows_per_worker,), jnp.int32),
            pltpu.VMEM((chunk, emb_dim), jnp.float32),
            pltpu.SemaphoreType.DMA,
        ],
    )
    def body(inp_hbm, table_hbm, out_hbm, row_v, idx_v, buf_v, sem):
        wid = lax.axis_index("c") * (n_workers // 2) + lax.axis_index("s")
        seg = wid % segs_per_row
        row = wid // segs_per_row
        row_start = row * seq_len

        pltpu.sync_copy(inp_hbm.at[pl.ds(row_start, seq_len)], row_v)

        def count_chunk(k, carry):
            x = row_v[pl.ds(k * L, L)]
            cs = _cumsum16(jnp.minimum(x, 1))
            return carry + _bcast_last(cs)

        zero = jnp.zeros((L,), jnp.int32)
        base = lax.fori_loop(0, seg * (rows_per_worker // L), count_chunk, zero)

        local0 = seg * rows_per_worker

        def pos_chunk(k, carry):
            x = row_v[pl.ds(local0 + k * L, L)]
            m = jnp.minimum(x, 1)
            cs = _cumsum16(m)
            idx_v[pl.ds(k * L, L)] = (carry + cs) * m
            return carry + _bcast_last(cs)

        lax.fori_loop(0, rows_per_worker // L, pos_chunk, base)

        out0 = wid * rows_per_worker

        def gather_chunk(i, _):
            idx_sl = idx_v.at[pl.ds(i * chunk, chunk)]
            pltpu.async_copy(table_hbm.at[idx_sl], buf_v, sem).wait()
            pltpu.sync_copy(buf_v, out_hbm.at[pl.ds(out0 + i * chunk, chunk)])
            return 0

        lax.fori_loop(0, n_chunks, gather_chunk, 0)

    return body(inp_flat, weights)


def kernel(input, weights):
    b, seq_len = input.shape
    inp_flat = input.reshape(-1)
    out = _emb_lookup(inp_flat, weights, rows_per_worker=(b * seq_len) // 32,
                      seq_len=seq_len, chunk=64)
    return out.reshape(b, seq_len, weights.shape[1])


# speedup vs baseline: 2.2191x; 1.0516x over previous
import functools

import jax
import jax.numpy as jnp
from jax import lax
from jax.experimental import pallas as pl
from jax.experimental.pallas import tpu as pltpu
from jax.experimental.pallas import tpu_sc as plsc

L = 16


def _take(v, idx):
    return v.at[idx].get(mode="promise_in_bounds")


def _cumsum16(m):
    lanes = lax.iota(jnp.int32, L)
    v = m
    for k in (1, 2, 4, 8):
        shifted = _take(v, jnp.maximum(lanes - k, 0))
        ind = jnp.minimum(jnp.maximum(lanes - (k - 1), 0), 1)
        v = v + shifted * ind
    return v


def _bcast_last(v):
    return _take(v, jnp.full((L,), L - 1, jnp.int32))


def _emb_lookup(inp_flat, weights, *, rows_per_worker, seq_len, chunk):
    n_workers = inp_flat.shape[0] // rows_per_worker
    segs_per_row = seq_len // rows_per_worker
    n_chunks = rows_per_worker // chunk
    emb_dim = weights.shape[1]
    mesh = plsc.VectorSubcoreMesh(core_axis_name="c", subcore_axis_name="s")

    @functools.partial(
        pl.kernel,
        out_type=jax.ShapeDtypeStruct((inp_flat.shape[0], emb_dim), jnp.float32),
        mesh=mesh,
        scratch_types=[
            pltpu.VMEM((seq_len,), jnp.int32),
            pltpu.VMEM((rows_per_worker,), jnp.int32),
            pltpu.VMEM((2, chunk, emb_dim), jnp.float32),
            pltpu.SemaphoreType.DMA,
            pltpu.SemaphoreType.DMA,
            pltpu.SemaphoreType.DMA,
            pltpu.SemaphoreType.DMA,
        ],
    )
    def body(inp_hbm, table_hbm, out_hbm, row_v, idx_v, buf_v, g0, g1, o0, o1):
        wid = lax.axis_index("c") * (n_workers // 2) + lax.axis_index("s")
        seg = wid % segs_per_row
        row = wid // segs_per_row
        row_start = row * seq_len

        pltpu.sync_copy(inp_hbm.at[pl.ds(row_start, seq_len)], row_v)

        def count_chunk(k, acc):
            x = row_v[pl.ds(k * L, L)]
            return acc + jnp.minimum(x, 1)

        zero = jnp.zeros((L,), jnp.int32)
        acc = lax.fori_loop(0, seg * (rows_per_worker // L), count_chunk, zero)
        base = _bcast_last(_cumsum16(acc))

        local0 = seg * rows_per_worker

        def pos_chunk(k, carry):
            x = row_v[pl.ds(local0 + k * L, L)]
            m = jnp.minimum(x, 1)
            cs = _cumsum16(m)
            idx_v[pl.ds(k * L, L)] = (carry + cs) * m
            return carry + _bcast_last(cs)

        lax.fori_loop(0, rows_per_worker // L, pos_chunk, base)

        out0 = wid * rows_per_worker
        gs, os_ = (g0, g1), (o0, o1)

        def start_gather(i, b):
            idx_sl = idx_v.at[pl.ds(i * chunk, chunk)]
            pltpu.async_copy(table_hbm.at[idx_sl], buf_v.at[b], gs[b])

        def wait_gather(b):
            pltpu.make_async_copy(
                table_hbm.at[pl.ds(0, chunk)], buf_v.at[b], gs[b]).wait()

        def start_out(i, b):
            pltpu.async_copy(
                buf_v.at[b], out_hbm.at[pl.ds(out0 + i * chunk, chunk)], os_[b])

        def wait_out(b):
            pltpu.make_async_copy(
                table_hbm.at[pl.ds(0, chunk)],
                out_hbm.at[pl.ds(out0, chunk)], os_[b]).wait()

        start_gather(0, 0)
        wait_gather(0)
        start_gather(1, 1)
        start_out(0, 0)

        def outer(i, _):
            b = i % 2
            wait_gather_dyn(b)
            wait_out_dyn(1 - b)
            start_gather_dyn(i + 1, 1 - b)
            start_out_dyn(i, b)
            return 0

        def wait_gather_dyn(b):
            @pl.when(b == 0)
            def _():
                wait_gather(0)
            @pl.when(b == 1)
            def _():
                wait_gather(1)

        def wait_out_dyn(b):
            @pl.when(b == 0)
            def _():
                wait_out(0)
            @pl.when(b == 1)
            def _():
                wait_out(1)

        def start_gather_dyn(i, b):
            @pl.when(b == 0)
            def _():
                start_gather(i, 0)
            @pl.when(b == 1)
            def _():
                start_gather(i, 1)

        def start_out_dyn(i, b):
            @pl.when(b == 0)
            def _():
                start_out(i, 0)
            @pl.when(b == 1)
            def _():
                start_out(i, 1)

        lax.fori_loop(1, n_chunks - 1, outer, 0)

        b_last = (n_chunks - 1) % 2
        wait_gather_dyn(b_last)
        wait_out_dyn(1 - b_last)
        start_out_dyn(n_chunks - 1, b_last)
        wait_out_dyn(b_last)

    return body(inp_flat, weights)


def kernel(input, weights):
    b, seq_len = input.shape
    inp_flat = input.reshape(-1)
    out = _emb_lookup(inp_flat, weights, rows_per_worker=(b * seq_len) // 32,
                      seq_len=seq_len, chunk=32)
    return out.reshape(b, seq_len, weights.shape[1])


# speedup vs baseline: 2.2204x; 1.0006x over previous
import functools

import jax
import jax.numpy as jnp
from jax import lax
from jax.experimental import pallas as pl
from jax.experimental.pallas import tpu as pltpu
from jax.experimental.pallas import tpu_sc as plsc

L = 16


def _take(v, idx):
    return v.at[idx].get(mode="promise_in_bounds")


def _cumsum16(m):
    lanes = lax.iota(jnp.int32, L)
    v = m
    for k in (1, 2, 4, 8):
        shifted = _take(v, jnp.maximum(lanes - k, 0))
        ind = jnp.minimum(jnp.maximum(lanes - (k - 1), 0), 1)
        v = v + shifted * ind
    return v


def _bcast_last(v):
    return _take(v, jnp.full((L,), L - 1, jnp.int32))


def _emb_lookup(inp_flat, weights, *, rows_per_worker, seq_len, chunk):
    n_workers = inp_flat.shape[0] // rows_per_worker
    segs_per_row = seq_len // rows_per_worker
    n_chunks = rows_per_worker // chunk
    emb_dim = weights.shape[1]
    mesh = plsc.VectorSubcoreMesh(core_axis_name="c", subcore_axis_name="s")

    @functools.partial(
        pl.kernel,
        out_type=jax.ShapeDtypeStruct((inp_flat.shape[0], emb_dim), jnp.float32),
        mesh=mesh,
        scratch_types=[
            pltpu.VMEM((seq_len,), jnp.int32),
            pltpu.VMEM((rows_per_worker,), jnp.int32),
            pltpu.VMEM((2, chunk, emb_dim), jnp.float32),
            pltpu.SemaphoreType.DMA,
            pltpu.SemaphoreType.DMA,
            pltpu.SemaphoreType.DMA,
            pltpu.SemaphoreType.DMA,
        ],
    )
    def body(inp_hbm, table_hbm, out_hbm, row_v, idx_v, buf_v, g0, g1, o0, o1):
        wid = lax.axis_index("c") * (n_workers // 2) + lax.axis_index("s")
        seg = wid % segs_per_row
        row = wid // segs_per_row
        row_start = row * seq_len

        pltpu.sync_copy(inp_hbm.at[pl.ds(row_start, seq_len)], row_v)

        def count_chunk(k, acc):
            x = row_v[pl.ds(k * L, L)]
            return acc + jnp.minimum(x, 1)

        zero = jnp.zeros((L,), jnp.int32)
        acc = lax.fori_loop(0, seg * (rows_per_worker // L), count_chunk, zero)
        base = _bcast_last(_cumsum16(acc))

        local0 = seg * rows_per_worker

        def pos_chunk(k, carry):
            x = row_v[pl.ds(local0 + k * L, L)]
            m = jnp.minimum(x, 1)
            cs = _cumsum16(m)
            idx_v[pl.ds(k * L, L)] = (carry + cs) * m
            return carry + _bcast_last(cs)

        lax.fori_loop(0, rows_per_worker // L, pos_chunk, base)

        out0 = wid * rows_per_worker
        gs, os_ = (g0, g1), (o0, o1)

        def start_gather(i, b):
            idx_sl = idx_v.at[pl.ds(i * chunk, chunk)]
            pltpu.async_copy(table_hbm.at[idx_sl], buf_v.at[b], gs[b])

        def wait_gather(b):
            pltpu.make_async_copy(
                table_hbm.at[pl.ds(0, chunk)], buf_v.at[b], gs[b]).wait()

        def start_out(i, b):
            pltpu.async_copy(
                buf_v.at[b], out_hbm.at[pl.ds(out0 + i * chunk, chunk)], os_[b])

        def wait_out(b):
            pltpu.make_async_copy(
                table_hbm.at[pl.ds(0, chunk)],
                out_hbm.at[pl.ds(out0, chunk)], os_[b]).wait()

        start_gather(0, 0)
        wait_gather(0)
        start_gather(1, 1)
        start_out(0, 0)

        def outer(j, _):
            i = 2 * j + 1
            wait_gather(1)
            wait_out(0)
            start_gather(i + 1, 0)
            start_out(i, 1)
            wait_gather(0)
            wait_out(1)
            start_gather(i + 2, 1)
            start_out(i + 1, 0)
            return 0

        lax.fori_loop(0, (n_chunks - 2) // 2, outer, 0)

        wait_gather(1)
        wait_out(0)
        start_out(n_chunks - 1, 1)
        wait_out(1)

    return body(inp_flat, weights)


def kernel(input, weights):
    b, seq_len = input.shape
    inp_flat = input.reshape(-1)
    out = _emb_lookup(inp_flat, weights, rows_per_worker=(b * seq_len) // 32,
                      seq_len=seq_len, chunk=32)
    return out.reshape(b, seq_len, weights.shape[1])
